# baseline (speedup 1.0000x reference)
import jax
import jax.numpy as jnp
from jax.experimental import pallas as pl
from jax.experimental.pallas import tpu as pltpu

_BLOCK_ROWS = 256
_LANES = 128


def _perm_mat(shift):
    row = jax.lax.broadcasted_iota(jnp.int32, (_LANES, _LANES), 0)
    col = jax.lax.broadcasted_iota(jnp.int32, (_LANES, _LANES), 1)
    src = (col & ~3) | ((col + shift) & 3)
    return jnp.where(row == src, 1.0, 0.0).astype(jnp.float32)


def _body(x_ref, o_ref):
    r, sub, lanes = x_ref.shape
    x = x_ref[...].reshape(r * sub, lanes)
    y1 = jax.lax.dot(x, _perm_mat(1), precision=jax.lax.Precision.HIGHEST)
    y2 = jax.lax.dot(x, _perm_mat(2), precision=jax.lax.Precision.HIGHEST)
    y3 = jax.lax.dot(x, _perm_mat(3), precision=jax.lax.Precision.HIGHEST)
    mx1 = jnp.maximum(x, y1)
    mn1 = jnp.minimum(x, y1)
    mx2 = jnp.maximum(y2, y3)
    mn2 = jnp.minimum(y2, y3)
    second = jnp.maximum(jnp.minimum(mx1, mx2), jnp.maximum(mn1, mn2))
    out = jnp.where(x >= second, x, jnp.zeros_like(x))
    o_ref[...] = out.reshape(r, sub, lanes)


def kernel(input):
    n, d = input.shape
    sub = d // _LANES
    x3 = input.reshape(n, sub, _LANES)
    grid = n // _BLOCK_ROWS
    out = pl.pallas_call(
        _body,
        grid=(grid,),
        in_specs=[pl.BlockSpec((_BLOCK_ROWS, sub, _LANES), lambda i: (i, 0, 0))],
        out_specs=pl.BlockSpec((_BLOCK_ROWS, sub, _LANES), lambda i: (i, 0, 0)),
        out_shape=jax.ShapeDtypeStruct((n, sub, _LANES), input.dtype),
        compiler_params=pltpu.CompilerParams(
            dimension_semantics=("arbitrary",),
        ),
    )(x3)
    return out.reshape(n, d)
